# baseline (speedup 1.0000x reference)
import functools
import math

import jax
import jax.numpy as jnp
from jax import lax
from jax.experimental import pallas as pl
from jax.experimental.pallas import tpu as pltpu

B = 2048
IN_DIM = 1024
N_SUPER = 2
N_SUB = 2
E = 4
HID = 2048
OUT = 512
AUX_COEF = 0.01

BT = 256
NT = B // BT

_HI = lax.Precision.HIGHEST


def _gating_body(x_ref, cgw1_ref, cgb1_ref, cgw2_ref, cgb2_ref,
                 fgw_ref, fgb_ref,
                 leaf_ref, aux_ref, acc_ref):
    i = pl.program_id(0)
    x = x_ref[...]
    h = lax.dot_general(x, cgw1_ref[...], (((1,), (1,)), ((), ())),
                        preferred_element_type=jnp.float32)
    h = jnp.maximum(h + cgb1_ref[...], 0.0)
    cl = lax.dot_general(h.astype(jnp.bfloat16), cgw2_ref[...],
                         (((1,), (1,)), ((), ())),
                         preferred_element_type=jnp.float32)
    cl = cl + cgb2_ref[...]
    m = jnp.max(cl, axis=1, keepdims=True)
    ex = jnp.exp(cl - m)
    cw = ex / jnp.sum(ex, axis=1, keepdims=True)
    ohc0 = (cl[:, 0:1] >= cl[:, 1:2]).astype(jnp.float32)

    x_aug = jnp.concatenate([x, cw.astype(jnp.bfloat16)], axis=1)
    fl = (lax.dot_general(x_aug, fgw_ref[...], (((1,), (1,)), ((), ())),
                          preferred_element_type=jnp.float32)
          + fgb_ref[...])
    oh0 = (fl[:, 0:1] >= fl[:, 1:2]).astype(jnp.float32)
    oh1 = (fl[:, 2:3] >= fl[:, 3:4]).astype(jnp.float32)

    m0 = jnp.maximum(fl[:, 0:1], fl[:, 1:2])
    e00 = jnp.exp(fl[:, 0:1] - m0)
    e01 = jnp.exp(fl[:, 1:2] - m0)
    p00 = e00 / (e00 + e01)
    m1 = jnp.maximum(fl[:, 2:3], fl[:, 3:4])
    e10 = jnp.exp(fl[:, 2:3] - m1)
    e11 = jnp.exp(fl[:, 3:4] - m1)
    p10 = e10 / (e10 + e11)

    c0 = cw[:, 0:1] * oh0
    c1 = cw[:, 0:1] * (1.0 - oh0)
    c2 = cw[:, 1:2] * oh1
    c3 = cw[:, 1:2] * (1.0 - oh1)
    den = (cw[:, 0:1] + cw[:, 1:2]) + 1e-8
    leaf_ref[...] = jnp.concatenate([c0, c1, c2, c3], axis=1) / den

    @pl.when(i == 0)
    def _init():
        for j in range(8):
            acc_ref[j] = 0.0

    acc_ref[0] += jnp.sum(ohc0)
    acc_ref[1] += jnp.sum(cw[:, 0:1])
    acc_ref[2] += jnp.sum(oh0)
    acc_ref[3] += jnp.sum(p00)
    acc_ref[4] += jnp.sum(oh1)
    acc_ref[5] += jnp.sum(p10)

    @pl.when(i == 0)
    def _zero_aux():
        aux_ref[...] = jnp.zeros((1, 1), jnp.float32)

    @pl.when(i == NT - 1)
    def _finalize():
        nb = jnp.float32(B)
        fc0 = acc_ref[0] / nb
        pc0 = acc_ref[1] / nb
        aux_c = 2.0 * (fc0 * pc0 + (1.0 - fc0) * (1.0 - pc0))
        f00 = acc_ref[2] / nb
        p00s = acc_ref[3] / nb
        f10 = acc_ref[4] / nb
        p10s = acc_ref[5] / nb
        aux_f = (2.0 * (f00 * p00s + (1.0 - f00) * (1.0 - p00s))
                 + 2.0 * (f10 * p10s + (1.0 - f10) * (1.0 - p10s)))
        aux_ref[...] = (AUX_COEF * (aux_c + aux_f / N_SUPER)).reshape(1, 1)


def _expert_body(leaf_ref, x_ref, w1_ref, b1_ref, w2_ref, b2_ref,
                 g_ref, beta_ref, rhw_ref, chw_ref, rhb_ref, chb_ref,
                 price_ref, dir_ref, pacc_ref, dacc_ref):
    e = pl.program_id(0)
    i = pl.program_id(1)
    x = x_ref[...]
    hh = lax.dot_general(x, w1_ref[0], (((1,), (1,)), ((), ())),
                         preferred_element_type=jnp.float32)
    hh = hh + b1_ref[0]
    hh = 0.5 * hh * (1.0 + lax.erf(hh * (1.0 / math.sqrt(2.0))))
    eo = lax.dot_general(hh.astype(jnp.bfloat16), w2_ref[0],
                         (((1,), (1,)), ((), ())),
                         preferred_element_type=jnp.float32)
    eo = eo + b2_ref[0]
    mu = jnp.mean(eo, axis=1, keepdims=True)
    d = eo - mu
    var = jnp.mean(d * d, axis=1, keepdims=True)
    rstd = lax.rsqrt(var + 1e-5)
    eon = d * rstd * g_ref[0] + beta_ref[0]
    pr = lax.dot_general(eon, rhw_ref[...], (((1,), (1,)), ((), ())),
                         preferred_element_type=jnp.float32)
    dr = lax.dot_general(eon, chw_ref[...], (((1,), (1,)), ((), ())),
                         preferred_element_type=jnp.float32)
    lane = lax.broadcasted_iota(jnp.int32, (1, E), 1)
    l = jnp.sum(jnp.where(lane == e, leaf_ref[...], 0.0), axis=1,
                keepdims=True)
    cp = l * pr
    cd = l * dr
    sl = pl.ds(i * BT, BT)

    @pl.when(e == 0)
    def _init():
        pacc_ref[sl, :] = cp
        dacc_ref[sl, :] = cd

    @pl.when(e > 0)
    def _acc():
        pacc_ref[sl, :] += cp
        dacc_ref[sl, :] += cd

    price_ref[...] = pacc_ref[sl, :] + rhb_ref[...]
    dir_ref[...] = 1.0 / (1.0 + jnp.exp(-(dacc_ref[sl, :] + chb_ref[...])))


@jax.jit
def kernel(x, cg_w1, cg_b1, cg_w2, cg_b2, fg_w, fg_b, ex_w1, ex_b1,
           ex_w2, ex_b2, ex_g, ex_beta, rh_w, rh_b, ch_w, ch_b):
    f32 = jnp.float32
    bf16 = jnp.bfloat16
    xb = x.astype(bf16)
    fg_w2d = fg_w.reshape(E, IN_DIM + N_SUPER).astype(bf16)
    leaf, aux = pl.pallas_call(
        _gating_body,
        grid=(NT,),
        in_specs=[
            pl.BlockSpec((BT, IN_DIM), lambda i: (i, 0)),
            pl.BlockSpec((IN_DIM // 2, IN_DIM), lambda i: (0, 0)),
            pl.BlockSpec((1, IN_DIM // 2), lambda i: (0, 0)),
            pl.BlockSpec((N_SUPER, IN_DIM // 2), lambda i: (0, 0)),
            pl.BlockSpec((1, N_SUPER), lambda i: (0, 0)),
            pl.BlockSpec((E, IN_DIM + N_SUPER), lambda i: (0, 0)),
            pl.BlockSpec((1, E), lambda i: (0, 0)),
        ],
        out_specs=[
            pl.BlockSpec((BT, E), lambda i: (i, 0)),
            pl.BlockSpec((1, 1), lambda i: (0, 0)),
        ],
        out_shape=[
            jax.ShapeDtypeStruct((B, E), f32),
            jax.ShapeDtypeStruct((1, 1), f32),
        ],
        scratch_shapes=[pltpu.SMEM((8,), f32)],
    )(xb, cg_w1.astype(bf16), cg_b1.reshape(1, -1),
      cg_w2.astype(bf16), cg_b2.reshape(1, -1),
      fg_w2d, fg_b.reshape(1, E))

    price, direction = pl.pallas_call(
        _expert_body,
        grid=(E, NT),
        in_specs=[
            pl.BlockSpec((BT, E), lambda e, i: (i, 0)),
            pl.BlockSpec((BT, IN_DIM), lambda e, i: (i, 0)),
            pl.BlockSpec((1, HID, IN_DIM), lambda e, i: (e, 0, 0)),
            pl.BlockSpec((1, 1, HID), lambda e, i: (e, 0, 0)),
            pl.BlockSpec((1, OUT, HID), lambda e, i: (e, 0, 0)),
            pl.BlockSpec((1, 1, OUT), lambda e, i: (e, 0, 0)),
            pl.BlockSpec((1, 1, OUT), lambda e, i: (e, 0, 0)),
            pl.BlockSpec((1, 1, OUT), lambda e, i: (e, 0, 0)),
            pl.BlockSpec((1, OUT), lambda e, i: (0, 0)),
            pl.BlockSpec((1, OUT), lambda e, i: (0, 0)),
            pl.BlockSpec((1, 1), lambda e, i: (0, 0)),
            pl.BlockSpec((1, 1), lambda e, i: (0, 0)),
        ],
        out_specs=[
            pl.BlockSpec((BT, 1), lambda e, i: (i, 0)),
            pl.BlockSpec((BT, 1), lambda e, i: (i, 0)),
        ],
        out_shape=[
            jax.ShapeDtypeStruct((B, 1), f32),
            jax.ShapeDtypeStruct((B, 1), f32),
        ],
        scratch_shapes=[
            pltpu.VMEM((B, 1), f32),
            pltpu.VMEM((B, 1), f32),
        ],
    )(leaf, xb, ex_w1.astype(bf16), ex_b1.reshape(E, 1, HID),
      ex_w2.astype(bf16),
      ex_b2.reshape(E, 1, OUT), ex_g.reshape(E, 1, OUT),
      ex_beta.reshape(E, 1, OUT),
      rh_w, ch_w, rh_b.reshape(1, 1), ch_b.reshape(1, 1))

    return price, direction, leaf, aux.reshape(())


# speedup vs baseline: 1.2001x; 1.2001x over previous
import functools
import math

import jax
import jax.numpy as jnp
from jax import lax
from jax.experimental import pallas as pl
from jax.experimental.pallas import tpu as pltpu
from jax.experimental.pallas import tpu_sc as plsc

B = 2048
IN_DIM = 1024
N_SUPER = 2
N_SUB = 2
E = 4
HID = 2048
OUT = 512
AUX_COEF = 0.01

BT = 256
NT = B // BT
LP = B + BT
NSLOT = 2 * LP
NTE = NSLOT // BT

NW = 32
TPW = B // NW

_mesh = plsc.VectorSubcoreMesh(core_axis_name="c", subcore_axis_name="s")


def _gating_body(x_ref, cgw1_ref, cgb1_ref, cgw2_ref, cgb2_ref,
                 fgw_ref, fgb_ref,
                 leaf_ref, aux_ref, w0_ref, w1_ref, pos0_ref, pos1_ref,
                 eot_ref, acc_ref, ab0_ref, ab1_ref, cum00_ref, cum10_ref):
    i = pl.program_id(0)
    x = x_ref[...]
    h = lax.dot_general(x, cgw1_ref[...], (((1,), (1,)), ((), ())),
                        preferred_element_type=jnp.float32)
    h = jnp.maximum(h + cgb1_ref[...], 0.0)
    cl = lax.dot_general(h, cgw2_ref[...], (((1,), (1,)), ((), ())),
                         preferred_element_type=jnp.float32)
    cl = cl + cgb2_ref[...]
    m = jnp.max(cl, axis=1, keepdims=True)
    ex = jnp.exp(cl - m)
    cw = ex / jnp.sum(ex, axis=1, keepdims=True)
    ohc0 = (cl[:, 0:1] >= cl[:, 1:2]).astype(jnp.float32)

    x_aug = jnp.concatenate([x, cw], axis=1)
    fl = (lax.dot_general(x_aug, fgw_ref[...], (((1,), (1,)), ((), ())),
                          preferred_element_type=jnp.float32)
          + fgb_ref[...])
    oh0 = (fl[:, 0:1] >= fl[:, 1:2]).astype(jnp.float32)
    oh1 = (fl[:, 2:3] >= fl[:, 3:4]).astype(jnp.float32)

    m0 = jnp.maximum(fl[:, 0:1], fl[:, 1:2])
    p00 = jnp.exp(fl[:, 0:1] - m0) / (jnp.exp(fl[:, 0:1] - m0)
                                      + jnp.exp(fl[:, 1:2] - m0))
    m1 = jnp.maximum(fl[:, 2:3], fl[:, 3:4])
    p10 = jnp.exp(fl[:, 2:3] - m1) / (jnp.exp(fl[:, 2:3] - m1)
                                      + jnp.exp(fl[:, 3:4] - m1))

    c0 = cw[:, 0:1] * oh0
    c1 = cw[:, 0:1] * (1.0 - oh0)
    c2 = cw[:, 1:2] * oh1
    c3 = cw[:, 1:2] * (1.0 - oh1)
    den = (cw[:, 0:1] + cw[:, 1:2]) + 1e-8
    leaf_ref[...] = jnp.concatenate([c0, c1, c2, c3], axis=1) / den
    w0_ref[...] = cw[:, 0:1] / den
    w1_ref[...] = cw[:, 1:2] / den
    sl = pl.ds(i * BT, BT)
    ab0_ref[sl, :] = 1.0 - oh0
    ab1_ref[sl, :] = 1.0 - oh1

    tri = (lax.broadcasted_iota(jnp.int32, (BT, BT), 0)
           >= lax.broadcasted_iota(jnp.int32, (BT, BT), 1)).astype(
               jnp.float32)
    tc0 = lax.dot_general(tri, oh0, (((1,), (0,)), ((), ())),
                          preferred_element_type=jnp.float32)
    tc1 = lax.dot_general(tri, oh1, (((1,), (0,)), ((), ())),
                          preferred_element_type=jnp.float32)

    @pl.when(i == 0)
    def _init():
        for j in range(8):
            acc_ref[j] = 0.0

    cum00_ref[sl, :] = tc0 + acc_ref[6]
    cum10_ref[sl, :] = tc1 + acc_ref[7]
    acc_ref[6] += jnp.sum(oh0)
    acc_ref[7] += jnp.sum(oh1)

    acc_ref[0] += jnp.sum(ohc0)
    acc_ref[1] += jnp.sum(cw[:, 0:1])
    acc_ref[2] += jnp.sum(oh0)
    acc_ref[3] += jnp.sum(p00)
    acc_ref[4] += jnp.sum(oh1)
    acc_ref[5] += jnp.sum(p10)

    @pl.when(i == NT - 1)
    def _finalize():
        nb = jnp.float32(B)
        fc0 = acc_ref[0] / nb
        pc0 = acc_ref[1] / nb
        aux_c = 2.0 * (fc0 * pc0 + (1.0 - fc0) * (1.0 - pc0))
        f00 = acc_ref[2] / nb
        p00s = acc_ref[3] / nb
        f10 = acc_ref[4] / nb
        p10s = acc_ref[5] / nb
        aux_f = (2.0 * (f00 * p00s + (1.0 - f00) * (1.0 - p00s))
                 + 2.0 * (f10 * p10s + (1.0 - f10) * (1.0 - p10s)))
        aux_ref[...] = (AUX_COEF * (aux_c + aux_f / N_SUPER)).reshape(1, 1)

        tglob1 = (lax.broadcasted_iota(jnp.int32, (B, 1), 0)
                  .astype(jnp.float32) + 1.0)

        def route(ab, cum_ref, n0):
            a = ab[...]
            cum0 = cum_ref[...]
            n0p = jnp.floor((n0 + (BT - 1)) * (1.0 / BT)) * BT
            cum1 = tglob1 - cum0
            pos = jnp.where(a == 0.0, cum0 - 1.0, n0p + cum1 - 1.0)
            return pos, n0p

        pos0, n0p0 = route(ab0_ref, cum00_ref, acc_ref[6])
        pos1, n0p1 = route(ab1_ref, cum10_ref, acc_ref[7])
        pos0_ref[...] = pos0.astype(jnp.int32)
        pos1_ref[...] = (pos1 + LP).astype(jnp.int32)

        t = lax.broadcasted_iota(jnp.int32, (1, 32), 1).astype(jnp.float32)
        e_g0 = jnp.where(t * BT < n0p0, 0.0, 1.0)
        e_g1 = jnp.where((t - NTE // 2) * BT < n0p1, 2.0, 3.0)
        eot = jnp.where(t < NTE // 2, e_g0,
                        jnp.where(t < NTE, e_g1, 0.0))
        eot_ref[...] = eot.astype(jnp.int32)


@functools.partial(
    pl.kernel, mesh=_mesh,
    out_type=jax.ShapeDtypeStruct((NSLOT, IN_DIM), jnp.float32),
    scratch_types=[
        pltpu.VMEM((TPW, IN_DIM), jnp.float32),
        pltpu.VMEM((TPW,), jnp.int32),
        pltpu.VMEM((TPW,), jnp.int32),
        pltpu.SemaphoreType.DMA,
    ],
)
def _sc_scatter(x_hbm, pos0_hbm, pos1_hbm, xp_hbm,
                rows_v, idx0_v, idx1_v, sem):
    wid = lax.axis_index("s") * 2 + lax.axis_index("c")
    base = wid * TPW
    pltpu.sync_copy(x_hbm.at[pl.ds(base, TPW)], rows_v)
    pltpu.sync_copy(pos0_hbm.at[pl.ds(base, TPW)], idx0_v)
    pltpu.sync_copy(pos1_hbm.at[pl.ds(base, TPW)], idx1_v)
    pltpu.async_copy(rows_v, xp_hbm.at[idx0_v], sem).wait()
    pltpu.async_copy(rows_v, xp_hbm.at[idx1_v], sem).wait()


def _expert_body(eot_ref, xp_ref, w1_ref, b1_ref, w2_ref, b2_ref,
                 g_ref, beta_ref, rhw_ref, chw_ref, prs_ref, drs_ref):
    x = xp_ref[...]
    hh = lax.dot_general(x, w1_ref[0], (((1,), (1,)), ((), ())),
                         preferred_element_type=jnp.float32)
    hh = hh + b1_ref[0]
    hh = 0.5 * hh * (1.0 + lax.erf(hh * (1.0 / math.sqrt(2.0))))
    eo = lax.dot_general(hh, w2_ref[0], (((1,), (1,)), ((), ())),
                         preferred_element_type=jnp.float32)
    eo = eo + b2_ref[0]
    mu = jnp.mean(eo, axis=1, keepdims=True)
    d = eo - mu
    var = jnp.mean(d * d, axis=1, keepdims=True)
    rstd = lax.rsqrt(var + 1e-5)
    eon = d * rstd * g_ref[0] + beta_ref[0]
    prs_ref[...] = lax.dot_general(eon, rhw_ref[...], (((1,), (1,)), ((), ())),
                                   preferred_element_type=jnp.float32)
    drs_ref[...] = lax.dot_general(eon, chw_ref[...], (((1,), (1,)), ((), ())),
                                   preferred_element_type=jnp.float32)


@functools.partial(
    pl.kernel, mesh=_mesh,
    out_type=[
        jax.ShapeDtypeStruct((B,), jnp.float32),
        jax.ShapeDtypeStruct((B,), jnp.float32),
    ],
    scratch_types=[
        pltpu.VMEM((TPW,), jnp.int32),
        pltpu.VMEM((TPW,), jnp.int32),
        pltpu.VMEM((TPW,), jnp.float32),
        pltpu.VMEM((TPW,), jnp.float32),
        pltpu.VMEM((32,), jnp.float32),
        pltpu.VMEM((TPW,), jnp.float32),
        pltpu.VMEM((TPW,), jnp.float32),
        pltpu.VMEM((TPW,), jnp.float32),
        pltpu.VMEM((TPW,), jnp.float32),
        pltpu.VMEM((TPW,), jnp.float32),
        pltpu.VMEM((TPW,), jnp.float32),
        pltpu.SemaphoreType.DMA,
    ],
)
def _sc_combine(prs_hbm, drs_hbm, pos0_hbm, pos1_hbm, w0_hbm, w1_hbm,
                bias_hbm, price_hbm, dir_hbm,
                idx0_v, idx1_v, w0_v, w1_v, bias_v,
                p0_v, p1_v, d0_v, d1_v, pout_v, dout_v, sem):
    wid = lax.axis_index("s") * 2 + lax.axis_index("c")
    base = wid * TPW
    pltpu.sync_copy(pos0_hbm.at[pl.ds(base, TPW)], idx0_v)
    pltpu.sync_copy(pos1_hbm.at[pl.ds(base, TPW)], idx1_v)
    pltpu.sync_copy(w0_hbm.at[pl.ds(base, TPW)], w0_v)
    pltpu.sync_copy(w1_hbm.at[pl.ds(base, TPW)], w1_v)
    pltpu.sync_copy(bias_hbm, bias_v)
    pltpu.async_copy(prs_hbm.at[idx0_v], p0_v, sem).wait()
    pltpu.async_copy(prs_hbm.at[idx1_v], p1_v, sem).wait()
    pltpu.async_copy(drs_hbm.at[idx0_v], d0_v, sem).wait()
    pltpu.async_copy(drs_hbm.at[idx1_v], d1_v, sem).wait()
    rb = bias_v[pl.ds(0, 16)]
    cb = bias_v[pl.ds(16, 16)]
    for j in range(TPW // 16):
        sl = pl.ds(j * 16, 16)
        a = w0_v[sl]
        bw = w1_v[sl]
        pout_v[sl] = a * p0_v[sl] + bw * p1_v[sl] + rb
        z = a * d0_v[sl] + bw * d1_v[sl] + cb
        dout_v[sl] = 1.0 / (1.0 + jnp.exp(-z))
    pltpu.sync_copy(pout_v, price_hbm.at[pl.ds(base, TPW)])
    pltpu.sync_copy(dout_v, dir_hbm.at[pl.ds(base, TPW)])


@jax.jit
def kernel(x, cg_w1, cg_b1, cg_w2, cg_b2, fg_w, fg_b, ex_w1, ex_b1,
           ex_w2, ex_b2, ex_g, ex_beta, rh_w, rh_b, ch_w, ch_b):
    f32 = jnp.float32
    fg_w2d = fg_w.reshape(E, IN_DIM + N_SUPER)
    leaf, aux, w0, w1, pos0, pos1, eot = pl.pallas_call(
        _gating_body,
        grid=(NT,),
        in_specs=[
            pl.BlockSpec((BT, IN_DIM), lambda i: (i, 0)),
            pl.BlockSpec((IN_DIM // 2, IN_DIM), lambda i: (0, 0)),
            pl.BlockSpec((1, IN_DIM // 2), lambda i: (0, 0)),
            pl.BlockSpec((N_SUPER, IN_DIM // 2), lambda i: (0, 0)),
            pl.BlockSpec((1, N_SUPER), lambda i: (0, 0)),
            pl.BlockSpec((E, IN_DIM + N_SUPER), lambda i: (0, 0)),
            pl.BlockSpec((1, E), lambda i: (0, 0)),
        ],
        out_specs=[
            pl.BlockSpec((BT, E), lambda i: (i, 0)),
            pl.BlockSpec((1, 1), lambda i: (0, 0)),
            pl.BlockSpec((BT, 1), lambda i: (i, 0)),
            pl.BlockSpec((BT, 1), lambda i: (i, 0)),
            pl.BlockSpec((B, 1), lambda i: (0, 0)),
            pl.BlockSpec((B, 1), lambda i: (0, 0)),
            pl.BlockSpec((1, 32), lambda i: (0, 0)),
        ],
        out_shape=[
            jax.ShapeDtypeStruct((B, E), f32),
            jax.ShapeDtypeStruct((1, 1), f32),
            jax.ShapeDtypeStruct((B, 1), f32),
            jax.ShapeDtypeStruct((B, 1), f32),
            jax.ShapeDtypeStruct((B, 1), jnp.int32),
            jax.ShapeDtypeStruct((B, 1), jnp.int32),
            jax.ShapeDtypeStruct((1, 32), jnp.int32),
        ],
        scratch_shapes=[
            pltpu.SMEM((8,), f32),
            pltpu.VMEM((B, 1), f32),
            pltpu.VMEM((B, 1), f32),
            pltpu.VMEM((B, 1), f32),
            pltpu.VMEM((B, 1), f32),
        ],
    )(x, cg_w1, cg_b1.reshape(1, -1), cg_w2, cg_b2.reshape(1, -1),
      fg_w2d, fg_b.reshape(1, E))

    pos0_1 = pos0.reshape(B)
    pos1_1 = pos1.reshape(B)
    x_perm = _sc_scatter(x, pos0_1, pos1_1)

    grid_spec = pltpu.PrefetchScalarGridSpec(
        num_scalar_prefetch=1,
        grid=(NTE,),
        in_specs=[
            pl.BlockSpec((BT, IN_DIM), lambda t, eot: (t, 0)),
            pl.BlockSpec((1, HID, IN_DIM), lambda t, eot: (eot[t], 0, 0)),
            pl.BlockSpec((1, 1, HID), lambda t, eot: (eot[t], 0, 0)),
            pl.BlockSpec((1, OUT, HID), lambda t, eot: (eot[t], 0, 0)),
            pl.BlockSpec((1, 1, OUT), lambda t, eot: (eot[t], 0, 0)),
            pl.BlockSpec((1, 1, OUT), lambda t, eot: (eot[t], 0, 0)),
            pl.BlockSpec((1, 1, OUT), lambda t, eot: (eot[t], 0, 0)),
            pl.BlockSpec((1, OUT), lambda t, eot: (0, 0)),
            pl.BlockSpec((1, OUT), lambda t, eot: (0, 0)),
        ],
        out_specs=[
            pl.BlockSpec((BT, 1), lambda t, eot: (t, 0)),
            pl.BlockSpec((BT, 1), lambda t, eot: (t, 0)),
        ],
    )
    prs, drs = pl.pallas_call(
        _expert_body,
        grid_spec=grid_spec,
        out_shape=[
            jax.ShapeDtypeStruct((NSLOT, 1), f32),
            jax.ShapeDtypeStruct((NSLOT, 1), f32),
        ],
    )(eot.reshape(32), x_perm, ex_w1, ex_b1.reshape(E, 1, HID), ex_w2,
      ex_b2.reshape(E, 1, OUT), ex_g.reshape(E, 1, OUT),
      ex_beta.reshape(E, 1, OUT), rh_w, ch_w)

    bias_arr = jnp.concatenate([
        jnp.broadcast_to(rh_b.reshape(1), (16,)),
        jnp.broadcast_to(ch_b.reshape(1), (16,)),
    ]).astype(f32)
    price, direction = _sc_combine(
        prs.reshape(NSLOT), drs.reshape(NSLOT), pos0_1, pos1_1,
        w0.reshape(B), w1.reshape(B), bias_arr)

    return price.reshape(B, 1), direction.reshape(B, 1), leaf, aux.reshape(())


# speedup vs baseline: 1.2674x; 1.0561x over previous
import functools
import math

import jax
import jax.numpy as jnp
from jax import lax
from jax.experimental import pallas as pl
from jax.experimental.pallas import tpu as pltpu
from jax.experimental.pallas import tpu_sc as plsc

B = 2048
IN_DIM = 1024
N_SUPER = 2
N_SUB = 2
E = 4
HID = 2048
OUT = 512
AUX_COEF = 0.01

BT = 256
NT = B // BT
BTE = 512
LP = B + BTE
NSLOT = 2 * LP
NTE = NSLOT // BTE

NW = 32
TPW = B // NW

_mesh = plsc.VectorSubcoreMesh(core_axis_name="c", subcore_axis_name="s")


def _gating_body(x_ref, cgw1_ref, cgb1_ref, cgw2_ref, cgb2_ref,
                 fgw_ref, fgb_ref,
                 leaf_ref, aux_ref, w0_ref, w1_ref, pos0_ref, pos1_ref,
                 eot_ref, acc_ref, ab0_ref, ab1_ref, cum00_ref, cum10_ref):
    i = pl.program_id(0)
    x = x_ref[...]
    h = lax.dot_general(x, cgw1_ref[...], (((1,), (1,)), ((), ())),
                        preferred_element_type=jnp.float32)
    h = jnp.maximum(h + cgb1_ref[...], 0.0)
    cl = lax.dot_general(h, cgw2_ref[...], (((1,), (1,)), ((), ())),
                         preferred_element_type=jnp.float32)
    cl = cl + cgb2_ref[...]
    m = jnp.max(cl, axis=1, keepdims=True)
    ex = jnp.exp(cl - m)
    cw = ex / jnp.sum(ex, axis=1, keepdims=True)
    ohc0 = (cl[:, 0:1] >= cl[:, 1:2]).astype(jnp.float32)

    x_aug = jnp.concatenate([x, cw], axis=1)
    fl = (lax.dot_general(x_aug, fgw_ref[...], (((1,), (1,)), ((), ())),
                          preferred_element_type=jnp.float32)
          + fgb_ref[...])
    oh0 = (fl[:, 0:1] >= fl[:, 1:2]).astype(jnp.float32)
    oh1 = (fl[:, 2:3] >= fl[:, 3:4]).astype(jnp.float32)

    m0 = jnp.maximum(fl[:, 0:1], fl[:, 1:2])
    p00 = jnp.exp(fl[:, 0:1] - m0) / (jnp.exp(fl[:, 0:1] - m0)
                                      + jnp.exp(fl[:, 1:2] - m0))
    m1 = jnp.maximum(fl[:, 2:3], fl[:, 3:4])
    p10 = jnp.exp(fl[:, 2:3] - m1) / (jnp.exp(fl[:, 2:3] - m1)
                                      + jnp.exp(fl[:, 3:4] - m1))

    c0 = cw[:, 0:1] * oh0
    c1 = cw[:, 0:1] * (1.0 - oh0)
    c2 = cw[:, 1:2] * oh1
    c3 = cw[:, 1:2] * (1.0 - oh1)
    den = (cw[:, 0:1] + cw[:, 1:2]) + 1e-8
    leaf_ref[...] = jnp.concatenate([c0, c1, c2, c3], axis=1) / den
    w0_ref[...] = cw[:, 0:1] / den
    w1_ref[...] = cw[:, 1:2] / den
    sl = pl.ds(i * BT, BT)
    ab0_ref[sl, :] = 1.0 - oh0
    ab1_ref[sl, :] = 1.0 - oh1

    tri = (lax.broadcasted_iota(jnp.int32, (BT, BT), 0)
           >= lax.broadcasted_iota(jnp.int32, (BT, BT), 1)).astype(
               jnp.float32)
    tc0 = lax.dot_general(tri, oh0, (((1,), (0,)), ((), ())),
                          preferred_element_type=jnp.float32)
    tc1 = lax.dot_general(tri, oh1, (((1,), (0,)), ((), ())),
                          preferred_element_type=jnp.float32)

    @pl.when(i == 0)
    def _init():
        for j in range(8):
            acc_ref[j] = 0.0

    cum00_ref[sl, :] = tc0 + acc_ref[6]
    cum10_ref[sl, :] = tc1 + acc_ref[7]
    acc_ref[6] += jnp.sum(oh0)
    acc_ref[7] += jnp.sum(oh1)

    acc_ref[0] += jnp.sum(ohc0)
    acc_ref[1] += jnp.sum(cw[:, 0:1])
    acc_ref[2] += jnp.sum(oh0)
    acc_ref[3] += jnp.sum(p00)
    acc_ref[4] += jnp.sum(oh1)
    acc_ref[5] += jnp.sum(p10)

    @pl.when(i == NT - 1)
    def _finalize():
        nb = jnp.float32(B)
        fc0 = acc_ref[0] / nb
        pc0 = acc_ref[1] / nb
        aux_c = 2.0 * (fc0 * pc0 + (1.0 - fc0) * (1.0 - pc0))
        f00 = acc_ref[2] / nb
        p00s = acc_ref[3] / nb
        f10 = acc_ref[4] / nb
        p10s = acc_ref[5] / nb
        aux_f = (2.0 * (f00 * p00s + (1.0 - f00) * (1.0 - p00s))
                 + 2.0 * (f10 * p10s + (1.0 - f10) * (1.0 - p10s)))
        aux_ref[...] = (AUX_COEF * (aux_c + aux_f / N_SUPER)).reshape(1, 1)

        tglob1 = (lax.broadcasted_iota(jnp.int32, (B, 1), 0)
                  .astype(jnp.float32) + 1.0)

        def route(ab, cum_ref, n0):
            a = ab[...]
            cum0 = cum_ref[...]
            n0p = jnp.floor((n0 + (BTE - 1)) * (1.0 / BTE)) * BTE
            cum1 = tglob1 - cum0
            pos = jnp.where(a == 0.0, cum0 - 1.0, n0p + cum1 - 1.0)
            return pos, n0p

        pos0, n0p0 = route(ab0_ref, cum00_ref, acc_ref[6])
        pos1, n0p1 = route(ab1_ref, cum10_ref, acc_ref[7])
        pos0_ref[...] = pos0.astype(jnp.int32)
        pos1_ref[...] = (pos1 + LP).astype(jnp.int32)

        t = lax.broadcasted_iota(jnp.int32, (1, 32), 1).astype(jnp.float32)
        e_g0 = jnp.where(t * BTE < n0p0, 0.0, 1.0)
        e_g1 = jnp.where((t - NTE // 2) * BTE < n0p1, 2.0, 3.0)
        eot = jnp.where(t < NTE // 2, e_g0,
                        jnp.where(t < NTE, e_g1, 0.0))
        eot_ref[...] = eot.astype(jnp.int32)


@functools.partial(
    pl.kernel, mesh=_mesh,
    out_type=jax.ShapeDtypeStruct((NSLOT, IN_DIM), jnp.float32),
    scratch_types=[
        pltpu.VMEM((TPW, IN_DIM), jnp.float32),
        pltpu.VMEM((TPW,), jnp.int32),
        pltpu.VMEM((TPW,), jnp.int32),
        pltpu.SemaphoreType.DMA,
    ],
)
def _sc_scatter(x_hbm, pos0_hbm, pos1_hbm, xp_hbm,
                rows_v, idx0_v, idx1_v, sem):
    wid = lax.axis_index("s") * 2 + lax.axis_index("c")
    base = wid * TPW
    pltpu.sync_copy(x_hbm.at[pl.ds(base, TPW)], rows_v)
    pltpu.sync_copy(pos0_hbm.at[pl.ds(base, TPW)], idx0_v)
    pltpu.sync_copy(pos1_hbm.at[pl.ds(base, TPW)], idx1_v)
    pltpu.async_copy(rows_v, xp_hbm.at[idx0_v], sem).wait()
    pltpu.async_copy(rows_v, xp_hbm.at[idx1_v], sem).wait()


def _expert_body(eot_ref, xp_ref, w1_ref, b1_ref, w2_ref, b2_ref,
                 g_ref, beta_ref, rhw_ref, chw_ref, prs_ref, drs_ref):
    x = xp_ref[...]
    hh = lax.dot_general(x, w1_ref[0], (((1,), (1,)), ((), ())),
                         preferred_element_type=jnp.float32)
    hh = hh + b1_ref[0]
    hh = 0.5 * hh * (1.0 + lax.erf(hh * (1.0 / math.sqrt(2.0))))
    eo = lax.dot_general(hh, w2_ref[0], (((1,), (1,)), ((), ())),
                         preferred_element_type=jnp.float32)
    eo = eo + b2_ref[0]
    mu = jnp.mean(eo, axis=1, keepdims=True)
    d = eo - mu
    var = jnp.mean(d * d, axis=1, keepdims=True)
    rstd = lax.rsqrt(var + 1e-5)
    eon = d * rstd * g_ref[0] + beta_ref[0]
    prs_ref[...] = lax.dot_general(eon, rhw_ref[...], (((1,), (1,)), ((), ())),
                                   preferred_element_type=jnp.float32)
    drs_ref[...] = lax.dot_general(eon, chw_ref[...], (((1,), (1,)), ((), ())),
                                   preferred_element_type=jnp.float32)


@functools.partial(
    pl.kernel, mesh=_mesh,
    out_type=[
        jax.ShapeDtypeStruct((B,), jnp.float32),
        jax.ShapeDtypeStruct((B,), jnp.float32),
    ],
    scratch_types=[
        pltpu.VMEM((TPW,), jnp.int32),
        pltpu.VMEM((TPW,), jnp.int32),
        pltpu.VMEM((TPW,), jnp.float32),
        pltpu.VMEM((TPW,), jnp.float32),
        pltpu.VMEM((32,), jnp.float32),
        pltpu.VMEM((TPW,), jnp.float32),
        pltpu.VMEM((TPW,), jnp.float32),
        pltpu.VMEM((TPW,), jnp.float32),
        pltpu.VMEM((TPW,), jnp.float32),
        pltpu.VMEM((TPW,), jnp.float32),
        pltpu.VMEM((TPW,), jnp.float32),
        pltpu.SemaphoreType.DMA,
    ],
)
def _sc_combine(prs_hbm, drs_hbm, pos0_hbm, pos1_hbm, w0_hbm, w1_hbm,
                bias_hbm, price_hbm, dir_hbm,
                idx0_v, idx1_v, w0_v, w1_v, bias_v,
                p0_v, p1_v, d0_v, d1_v, pout_v, dout_v, sem):
    wid = lax.axis_index("s") * 2 + lax.axis_index("c")
    base = wid * TPW
    pltpu.sync_copy(pos0_hbm.at[pl.ds(base, TPW)], idx0_v)
    pltpu.sync_copy(pos1_hbm.at[pl.ds(base, TPW)], idx1_v)
    pltpu.sync_copy(w0_hbm.at[pl.ds(base, TPW)], w0_v)
    pltpu.sync_copy(w1_hbm.at[pl.ds(base, TPW)], w1_v)
    pltpu.sync_copy(bias_hbm, bias_v)
    pltpu.async_copy(prs_hbm.at[idx0_v], p0_v, sem).wait()
    pltpu.async_copy(prs_hbm.at[idx1_v], p1_v, sem).wait()
    pltpu.async_copy(drs_hbm.at[idx0_v], d0_v, sem).wait()
    pltpu.async_copy(drs_hbm.at[idx1_v], d1_v, sem).wait()
    rb = bias_v[pl.ds(0, 16)]
    cb = bias_v[pl.ds(16, 16)]
    for j in range(TPW // 16):
        sl = pl.ds(j * 16, 16)
        a = w0_v[sl]
        bw = w1_v[sl]
        pout_v[sl] = a * p0_v[sl] + bw * p1_v[sl] + rb
        z = a * d0_v[sl] + bw * d1_v[sl] + cb
        dout_v[sl] = 1.0 / (1.0 + jnp.exp(-z))
    pltpu.sync_copy(pout_v, price_hbm.at[pl.ds(base, TPW)])
    pltpu.sync_copy(dout_v, dir_hbm.at[pl.ds(base, TPW)])


@jax.jit
def kernel(x, cg_w1, cg_b1, cg_w2, cg_b2, fg_w, fg_b, ex_w1, ex_b1,
           ex_w2, ex_b2, ex_g, ex_beta, rh_w, rh_b, ch_w, ch_b):
    f32 = jnp.float32
    fg_w2d = fg_w.reshape(E, IN_DIM + N_SUPER)
    leaf, aux, w0, w1, pos0, pos1, eot = pl.pallas_call(
        _gating_body,
        grid=(NT,),
        in_specs=[
            pl.BlockSpec((BT, IN_DIM), lambda i: (i, 0)),
            pl.BlockSpec((IN_DIM // 2, IN_DIM), lambda i: (0, 0)),
            pl.BlockSpec((1, IN_DIM // 2), lambda i: (0, 0)),
            pl.BlockSpec((N_SUPER, IN_DIM // 2), lambda i: (0, 0)),
            pl.BlockSpec((1, N_SUPER), lambda i: (0, 0)),
            pl.BlockSpec((E, IN_DIM + N_SUPER), lambda i: (0, 0)),
            pl.BlockSpec((1, E), lambda i: (0, 0)),
        ],
        out_specs=[
            pl.BlockSpec((BT, E), lambda i: (i, 0)),
            pl.BlockSpec((1, 1), lambda i: (0, 0)),
            pl.BlockSpec((BT, 1), lambda i: (i, 0)),
            pl.BlockSpec((BT, 1), lambda i: (i, 0)),
            pl.BlockSpec((B, 1), lambda i: (0, 0)),
            pl.BlockSpec((B, 1), lambda i: (0, 0)),
            pl.BlockSpec((1, 32), lambda i: (0, 0)),
        ],
        out_shape=[
            jax.ShapeDtypeStruct((B, E), f32),
            jax.ShapeDtypeStruct((1, 1), f32),
            jax.ShapeDtypeStruct((B, 1), f32),
            jax.ShapeDtypeStruct((B, 1), f32),
            jax.ShapeDtypeStruct((B, 1), jnp.int32),
            jax.ShapeDtypeStruct((B, 1), jnp.int32),
            jax.ShapeDtypeStruct((1, 32), jnp.int32),
        ],
        scratch_shapes=[
            pltpu.SMEM((8,), f32),
            pltpu.VMEM((B, 1), f32),
            pltpu.VMEM((B, 1), f32),
            pltpu.VMEM((B, 1), f32),
            pltpu.VMEM((B, 1), f32),
        ],
    )(x, cg_w1, cg_b1.reshape(1, -1), cg_w2, cg_b2.reshape(1, -1),
      fg_w2d, fg_b.reshape(1, E))

    pos0_1 = pos0.reshape(B)
    pos1_1 = pos1.reshape(B)
    x_perm = _sc_scatter(x, pos0_1, pos1_1)

    grid_spec = pltpu.PrefetchScalarGridSpec(
        num_scalar_prefetch=1,
        grid=(NTE,),
        in_specs=[
            pl.BlockSpec((BTE, IN_DIM), lambda t, eot: (t, 0)),
            pl.BlockSpec((1, HID, IN_DIM), lambda t, eot: (eot[t], 0, 0)),
            pl.BlockSpec((1, 1, HID), lambda t, eot: (eot[t], 0, 0)),
            pl.BlockSpec((1, OUT, HID), lambda t, eot: (eot[t], 0, 0)),
            pl.BlockSpec((1, 1, OUT), lambda t, eot: (eot[t], 0, 0)),
            pl.BlockSpec((1, 1, OUT), lambda t, eot: (eot[t], 0, 0)),
            pl.BlockSpec((1, 1, OUT), lambda t, eot: (eot[t], 0, 0)),
            pl.BlockSpec((1, OUT), lambda t, eot: (0, 0)),
            pl.BlockSpec((1, OUT), lambda t, eot: (0, 0)),
        ],
        out_specs=[
            pl.BlockSpec((BTE, 1), lambda t, eot: (t, 0)),
            pl.BlockSpec((BTE, 1), lambda t, eot: (t, 0)),
        ],
    )
    prs, drs = pl.pallas_call(
        _expert_body,
        grid_spec=grid_spec,
        out_shape=[
            jax.ShapeDtypeStruct((NSLOT, 1), f32),
            jax.ShapeDtypeStruct((NSLOT, 1), f32),
        ],
    )(eot.reshape(32), x_perm, ex_w1, ex_b1.reshape(E, 1, HID), ex_w2,
      ex_b2.reshape(E, 1, OUT), ex_g.reshape(E, 1, OUT),
      ex_beta.reshape(E, 1, OUT), rh_w, ch_w)

    bias_arr = jnp.concatenate([
        jnp.broadcast_to(rh_b.reshape(1), (16,)),
        jnp.broadcast_to(ch_b.reshape(1), (16,)),
    ]).astype(f32)
    price, direction = _sc_combine(
        prs.reshape(NSLOT), drs.reshape(NSLOT), pos0_1, pos1_1,
        w0.reshape(B), w1.reshape(B), bias_arr)

    return price.reshape(B, 1), direction.reshape(B, 1), leaf, aux.reshape(())


# speedup vs baseline: 1.2829x; 1.0122x over previous
import functools
import math

import jax
import jax.numpy as jnp
from jax import lax
from jax.experimental import pallas as pl
from jax.experimental.pallas import tpu as pltpu
from jax.experimental.pallas import tpu_sc as plsc

B = 2048
IN_DIM = 1024
N_SUPER = 2
N_SUB = 2
E = 4
HID = 2048
OUT = 512
AUX_COEF = 0.01

BT = 256
NT = B // BT
BTE = 512
LP = B + BTE
NSLOT = 2 * LP
NTE = NSLOT // BTE

NW = 32
TPW = B // NW

_mesh = plsc.VectorSubcoreMesh(core_axis_name="c", subcore_axis_name="s")


def _gating_body(x_ref, cgw1_ref, cgb1_ref, cgw2_ref, cgb2_ref,
                 fgw_ref, fgb_ref,
                 leaf_ref, aux_ref, w0_ref, w1_ref, pos0_ref, pos1_ref,
                 eot_ref, acc_ref, ab0_ref, ab1_ref, cum00_ref, cum10_ref):
    i = pl.program_id(0)
    x = x_ref[...]
    h = lax.dot_general(x, cgw1_ref[...], (((1,), (1,)), ((), ())),
                        preferred_element_type=jnp.float32)
    h = jnp.maximum(h + cgb1_ref[...], 0.0)
    cl = lax.dot_general(h, cgw2_ref[...], (((1,), (1,)), ((), ())),
                         preferred_element_type=jnp.float32)
    cl = cl + cgb2_ref[...]
    m = jnp.max(cl, axis=1, keepdims=True)
    ex = jnp.exp(cl - m)
    cw = ex / jnp.sum(ex, axis=1, keepdims=True)
    ohc0 = (cl[:, 0:1] >= cl[:, 1:2]).astype(jnp.float32)

    x_aug = jnp.concatenate([x, cw], axis=1)
    fl = (lax.dot_general(x_aug, fgw_ref[...], (((1,), (1,)), ((), ())),
                          preferred_element_type=jnp.float32)
          + fgb_ref[...])
    oh0 = (fl[:, 0:1] >= fl[:, 1:2]).astype(jnp.float32)
    oh1 = (fl[:, 2:3] >= fl[:, 3:4]).astype(jnp.float32)

    m0 = jnp.maximum(fl[:, 0:1], fl[:, 1:2])
    p00 = jnp.exp(fl[:, 0:1] - m0) / (jnp.exp(fl[:, 0:1] - m0)
                                      + jnp.exp(fl[:, 1:2] - m0))
    m1 = jnp.maximum(fl[:, 2:3], fl[:, 3:4])
    p10 = jnp.exp(fl[:, 2:3] - m1) / (jnp.exp(fl[:, 2:3] - m1)
                                      + jnp.exp(fl[:, 3:4] - m1))

    c0 = cw[:, 0:1] * oh0
    c1 = cw[:, 0:1] * (1.0 - oh0)
    c2 = cw[:, 1:2] * oh1
    c3 = cw[:, 1:2] * (1.0 - oh1)
    den = (cw[:, 0:1] + cw[:, 1:2]) + 1e-8
    leaf_ref[...] = jnp.concatenate([c0, c1, c2, c3], axis=1) / den
    w0_ref[...] = cw[:, 0:1] / den
    w1_ref[...] = cw[:, 1:2] / den
    sl = pl.ds(i * BT, BT)
    ab0_ref[sl, :] = 1.0 - oh0
    ab1_ref[sl, :] = 1.0 - oh1

    tri = (lax.broadcasted_iota(jnp.int32, (BT, BT), 0)
           >= lax.broadcasted_iota(jnp.int32, (BT, BT), 1)).astype(
               jnp.float32)
    tc0 = lax.dot_general(tri, oh0, (((1,), (0,)), ((), ())),
                          preferred_element_type=jnp.float32)
    tc1 = lax.dot_general(tri, oh1, (((1,), (0,)), ((), ())),
                          preferred_element_type=jnp.float32)

    @pl.when(i == 0)
    def _init():
        for j in range(8):
            acc_ref[j] = 0.0

    cum00_ref[sl, :] = tc0 + acc_ref[6]
    cum10_ref[sl, :] = tc1 + acc_ref[7]
    acc_ref[6] += jnp.sum(oh0)
    acc_ref[7] += jnp.sum(oh1)

    acc_ref[0] += jnp.sum(ohc0)
    acc_ref[1] += jnp.sum(cw[:, 0:1])
    acc_ref[2] += jnp.sum(oh0)
    acc_ref[3] += jnp.sum(p00)
    acc_ref[4] += jnp.sum(oh1)
    acc_ref[5] += jnp.sum(p10)

    @pl.when(i == NT - 1)
    def _finalize():
        nb = jnp.float32(B)
        fc0 = acc_ref[0] / nb
        pc0 = acc_ref[1] / nb
        aux_c = 2.0 * (fc0 * pc0 + (1.0 - fc0) * (1.0 - pc0))
        f00 = acc_ref[2] / nb
        p00s = acc_ref[3] / nb
        f10 = acc_ref[4] / nb
        p10s = acc_ref[5] / nb
        aux_f = (2.0 * (f00 * p00s + (1.0 - f00) * (1.0 - p00s))
                 + 2.0 * (f10 * p10s + (1.0 - f10) * (1.0 - p10s)))
        aux_ref[...] = (AUX_COEF * (aux_c + aux_f / N_SUPER)).reshape(1, 1)

        tglob1 = (lax.broadcasted_iota(jnp.int32, (B, 1), 0)
                  .astype(jnp.float32) + 1.0)

        def route(ab, cum_ref, n0):
            a = ab[...]
            cum0 = cum_ref[...]
            n0p = jnp.floor((n0 + (BTE - 1)) * (1.0 / BTE)) * BTE
            cum1 = tglob1 - cum0
            pos = jnp.where(a == 0.0, cum0 - 1.0, n0p + cum1 - 1.0)
            return pos, n0p

        pos0, n0p0 = route(ab0_ref, cum00_ref, acc_ref[6])
        pos1, n0p1 = route(ab1_ref, cum10_ref, acc_ref[7])
        pos0_ref[...] = pos0.astype(jnp.int32)
        pos1_ref[...] = (pos1 + LP).astype(jnp.int32)

        t = lax.broadcasted_iota(jnp.int32, (1, 32), 1).astype(jnp.float32)
        e_g0 = jnp.where(t * BTE < n0p0, 0.0, 1.0)
        e_g1 = jnp.where((t - NTE // 2) * BTE < n0p1, 2.0, 3.0)
        eot = jnp.where(t < NTE // 2, e_g0,
                        jnp.where(t < NTE, e_g1, 0.0))
        eot_ref[...] = eot.astype(jnp.int32)


@functools.partial(
    pl.kernel, mesh=_mesh,
    out_type=jax.ShapeDtypeStruct((NSLOT, IN_DIM), jnp.float32),
    scratch_types=[
        pltpu.VMEM((TPW, IN_DIM), jnp.float32),
        pltpu.VMEM((TPW,), jnp.int32),
        pltpu.VMEM((TPW,), jnp.int32),
        pltpu.SemaphoreType.DMA,
    ],
)
def _sc_scatter(x_hbm, pos0_hbm, pos1_hbm, xp_hbm,
                rows_v, idx0_v, idx1_v, sem):
    wid = lax.axis_index("s") * 2 + lax.axis_index("c")
    base = wid * TPW
    pltpu.sync_copy(x_hbm.at[pl.ds(base, TPW)], rows_v)
    pltpu.sync_copy(pos0_hbm.at[pl.ds(base, TPW)], idx0_v)
    pltpu.sync_copy(pos1_hbm.at[pl.ds(base, TPW)], idx1_v)
    c0 = pltpu.async_copy(rows_v, xp_hbm.at[idx0_v], sem)
    c1 = pltpu.async_copy(rows_v, xp_hbm.at[idx1_v], sem)
    c0.wait()
    c1.wait()


def _expert_body(eot_ref, xp_ref, w1_ref, b1_ref, w2_ref, b2_ref,
                 g_ref, beta_ref, rhw_ref, chw_ref, prs_ref, drs_ref):
    x = xp_ref[...]
    hh = lax.dot_general(x, w1_ref[0], (((1,), (1,)), ((), ())),
                         preferred_element_type=jnp.float32)
    hh = hh + b1_ref[0]
    hh = 0.5 * hh * (1.0 + lax.erf(hh * (1.0 / math.sqrt(2.0))))
    eo = lax.dot_general(hh, w2_ref[0], (((1,), (1,)), ((), ())),
                         preferred_element_type=jnp.float32)
    eo = eo + b2_ref[0]
    mu = jnp.mean(eo, axis=1, keepdims=True)
    d = eo - mu
    var = jnp.mean(d * d, axis=1, keepdims=True)
    rstd = lax.rsqrt(var + 1e-5)
    eon = d * rstd * g_ref[0] + beta_ref[0]
    prs_ref[...] = lax.dot_general(eon, rhw_ref[...], (((1,), (1,)), ((), ())),
                                   preferred_element_type=jnp.float32)
    drs_ref[...] = lax.dot_general(eon, chw_ref[...], (((1,), (1,)), ((), ())),
                                   preferred_element_type=jnp.float32)


@functools.partial(
    pl.kernel, mesh=_mesh,
    out_type=[
        jax.ShapeDtypeStruct((B,), jnp.float32),
        jax.ShapeDtypeStruct((B,), jnp.float32),
    ],
    scratch_types=[
        pltpu.VMEM((TPW,), jnp.int32),
        pltpu.VMEM((TPW,), jnp.int32),
        pltpu.VMEM((TPW,), jnp.float32),
        pltpu.VMEM((TPW,), jnp.float32),
        pltpu.VMEM((32,), jnp.float32),
        pltpu.VMEM((TPW,), jnp.float32),
        pltpu.VMEM((TPW,), jnp.float32),
        pltpu.VMEM((TPW,), jnp.float32),
        pltpu.VMEM((TPW,), jnp.float32),
        pltpu.VMEM((TPW,), jnp.float32),
        pltpu.VMEM((TPW,), jnp.float32),
        pltpu.SemaphoreType.DMA,
    ],
)
def _sc_combine(prs_hbm, drs_hbm, pos0_hbm, pos1_hbm, w0_hbm, w1_hbm,
                bias_hbm, price_hbm, dir_hbm,
                idx0_v, idx1_v, w0_v, w1_v, bias_v,
                p0_v, p1_v, d0_v, d1_v, pout_v, dout_v, sem):
    wid = lax.axis_index("s") * 2 + lax.axis_index("c")
    base = wid * TPW
    pltpu.sync_copy(pos0_hbm.at[pl.ds(base, TPW)], idx0_v)
    pltpu.sync_copy(pos1_hbm.at[pl.ds(base, TPW)], idx1_v)
    pltpu.sync_copy(w0_hbm.at[pl.ds(base, TPW)], w0_v)
    pltpu.sync_copy(w1_hbm.at[pl.ds(base, TPW)], w1_v)
    pltpu.sync_copy(bias_hbm, bias_v)
    g0 = pltpu.async_copy(prs_hbm.at[idx0_v], p0_v, sem)
    g1 = pltpu.async_copy(prs_hbm.at[idx1_v], p1_v, sem)
    g2 = pltpu.async_copy(drs_hbm.at[idx0_v], d0_v, sem)
    g3 = pltpu.async_copy(drs_hbm.at[idx1_v], d1_v, sem)
    g0.wait()
    g1.wait()
    g2.wait()
    g3.wait()
    rb = bias_v[pl.ds(0, 16)]
    cb = bias_v[pl.ds(16, 16)]
    for j in range(TPW // 16):
        sl = pl.ds(j * 16, 16)
        a = w0_v[sl]
        bw = w1_v[sl]
        pout_v[sl] = a * p0_v[sl] + bw * p1_v[sl] + rb
        z = a * d0_v[sl] + bw * d1_v[sl] + cb
        dout_v[sl] = 1.0 / (1.0 + jnp.exp(-z))
    pltpu.sync_copy(pout_v, price_hbm.at[pl.ds(base, TPW)])
    pltpu.sync_copy(dout_v, dir_hbm.at[pl.ds(base, TPW)])


@jax.jit
def kernel(x, cg_w1, cg_b1, cg_w2, cg_b2, fg_w, fg_b, ex_w1, ex_b1,
           ex_w2, ex_b2, ex_g, ex_beta, rh_w, rh_b, ch_w, ch_b):
    f32 = jnp.float32
    fg_w2d = fg_w.reshape(E, IN_DIM + N_SUPER)
    leaf, aux, w0, w1, pos0, pos1, eot = pl.pallas_call(
        _gating_body,
        grid=(NT,),
        in_specs=[
            pl.BlockSpec((BT, IN_DIM), lambda i: (i, 0)),
            pl.BlockSpec((IN_DIM // 2, IN_DIM), lambda i: (0, 0)),
            pl.BlockSpec((1, IN_DIM // 2), lambda i: (0, 0)),
            pl.BlockSpec((N_SUPER, IN_DIM // 2), lambda i: (0, 0)),
            pl.BlockSpec((1, N_SUPER), lambda i: (0, 0)),
            pl.BlockSpec((E, IN_DIM + N_SUPER), lambda i: (0, 0)),
            pl.BlockSpec((1, E), lambda i: (0, 0)),
        ],
        out_specs=[
            pl.BlockSpec((BT, E), lambda i: (i, 0)),
            pl.BlockSpec((1, 1), lambda i: (0, 0)),
            pl.BlockSpec((BT, 1), lambda i: (i, 0)),
            pl.BlockSpec((BT, 1), lambda i: (i, 0)),
            pl.BlockSpec((B, 1), lambda i: (0, 0)),
            pl.BlockSpec((B, 1), lambda i: (0, 0)),
            pl.BlockSpec((1, 32), lambda i: (0, 0)),
        ],
        out_shape=[
            jax.ShapeDtypeStruct((B, E), f32),
            jax.ShapeDtypeStruct((1, 1), f32),
            jax.ShapeDtypeStruct((B, 1), f32),
            jax.ShapeDtypeStruct((B, 1), f32),
            jax.ShapeDtypeStruct((B, 1), jnp.int32),
            jax.ShapeDtypeStruct((B, 1), jnp.int32),
            jax.ShapeDtypeStruct((1, 32), jnp.int32),
        ],
        scratch_shapes=[
            pltpu.SMEM((8,), f32),
            pltpu.VMEM((B, 1), f32),
            pltpu.VMEM((B, 1), f32),
            pltpu.VMEM((B, 1), f32),
            pltpu.VMEM((B, 1), f32),
        ],
    )(x, cg_w1, cg_b1.reshape(1, -1), cg_w2, cg_b2.reshape(1, -1),
      fg_w2d, fg_b.reshape(1, E))

    pos0_1 = pos0.reshape(B)
    pos1_1 = pos1.reshape(B)
    x_perm = _sc_scatter(x, pos0_1, pos1_1)

    grid_spec = pltpu.PrefetchScalarGridSpec(
        num_scalar_prefetch=1,
        grid=(NTE,),
        in_specs=[
            pl.BlockSpec((BTE, IN_DIM), lambda t, eot: (t, 0)),
            pl.BlockSpec((1, HID, IN_DIM), lambda t, eot: (eot[t], 0, 0)),
            pl.BlockSpec((1, 1, HID), lambda t, eot: (eot[t], 0, 0)),
            pl.BlockSpec((1, OUT, HID), lambda t, eot: (eot[t], 0, 0)),
            pl.BlockSpec((1, 1, OUT), lambda t, eot: (eot[t], 0, 0)),
            pl.BlockSpec((1, 1, OUT), lambda t, eot: (eot[t], 0, 0)),
            pl.BlockSpec((1, 1, OUT), lambda t, eot: (eot[t], 0, 0)),
            pl.BlockSpec((1, OUT), lambda t, eot: (0, 0)),
            pl.BlockSpec((1, OUT), lambda t, eot: (0, 0)),
        ],
        out_specs=[
            pl.BlockSpec((BTE, 1), lambda t, eot: (t, 0)),
            pl.BlockSpec((BTE, 1), lambda t, eot: (t, 0)),
        ],
    )
    prs, drs = pl.pallas_call(
        _expert_body,
        grid_spec=grid_spec,
        out_shape=[
            jax.ShapeDtypeStruct((NSLOT, 1), f32),
            jax.ShapeDtypeStruct((NSLOT, 1), f32),
        ],
    )(eot.reshape(32), x_perm, ex_w1, ex_b1.reshape(E, 1, HID), ex_w2,
      ex_b2.reshape(E, 1, OUT), ex_g.reshape(E, 1, OUT),
      ex_beta.reshape(E, 1, OUT), rh_w, ch_w)

    bias_arr = jnp.concatenate([
        jnp.broadcast_to(rh_b.reshape(1), (16,)),
        jnp.broadcast_to(ch_b.reshape(1), (16,)),
    ]).astype(f32)
    price, direction = _sc_combine(
        prs.reshape(NSLOT), drs.reshape(NSLOT), pos0_1, pos1_1,
        w0.reshape(B), w1.reshape(B), bias_arr)

    return price.reshape(B, 1), direction.reshape(B, 1), leaf, aux.reshape(())


# speedup vs baseline: 1.2955x; 1.0099x over previous
import functools
import math

import jax
import jax.numpy as jnp
from jax import lax
from jax.experimental import pallas as pl
from jax.experimental.pallas import tpu as pltpu
from jax.experimental.pallas import tpu_sc as plsc

B = 2048
IN_DIM = 1024
N_SUPER = 2
N_SUB = 2
E = 4
HID = 2048
OUT = 512
AUX_COEF = 0.01

BT = 512
NT = B // BT
BTE = 512
LP = B + BTE
NSLOT = 2 * LP
NTE = NSLOT // BTE

NW = 32
TPW = B // NW

_mesh = plsc.VectorSubcoreMesh(core_axis_name="c", subcore_axis_name="s")


def _gating_body(x_ref, cgw1_ref, cgb1_ref, cgw2_ref, cgb2_ref,
                 fgw_ref, fgb_ref,
                 leaf_ref, aux_ref, w0_ref, w1_ref, pos0_ref, pos1_ref,
                 eot_ref, acc_ref, ab0_ref, ab1_ref, cum00_ref, cum10_ref):
    i = pl.program_id(0)
    x = x_ref[...]
    h = lax.dot_general(x, cgw1_ref[...], (((1,), (1,)), ((), ())),
                        preferred_element_type=jnp.float32)
    h = jnp.maximum(h + cgb1_ref[...], 0.0)
    cl = lax.dot_general(h, cgw2_ref[...], (((1,), (1,)), ((), ())),
                         preferred_element_type=jnp.float32)
    cl = cl + cgb2_ref[...]
    m = jnp.max(cl, axis=1, keepdims=True)
    ex = jnp.exp(cl - m)
    cw = ex / jnp.sum(ex, axis=1, keepdims=True)
    ohc0 = (cl[:, 0:1] >= cl[:, 1:2]).astype(jnp.float32)

    x_aug = jnp.concatenate([x, cw], axis=1)
    fl = (lax.dot_general(x_aug, fgw_ref[...], (((1,), (1,)), ((), ())),
                          preferred_element_type=jnp.float32)
          + fgb_ref[...])
    oh0 = (fl[:, 0:1] >= fl[:, 1:2]).astype(jnp.float32)
    oh1 = (fl[:, 2:3] >= fl[:, 3:4]).astype(jnp.float32)

    m0 = jnp.maximum(fl[:, 0:1], fl[:, 1:2])
    p00 = jnp.exp(fl[:, 0:1] - m0) / (jnp.exp(fl[:, 0:1] - m0)
                                      + jnp.exp(fl[:, 1:2] - m0))
    m1 = jnp.maximum(fl[:, 2:3], fl[:, 3:4])
    p10 = jnp.exp(fl[:, 2:3] - m1) / (jnp.exp(fl[:, 2:3] - m1)
                                      + jnp.exp(fl[:, 3:4] - m1))

    c0 = cw[:, 0:1] * oh0
    c1 = cw[:, 0:1] * (1.0 - oh0)
    c2 = cw[:, 1:2] * oh1
    c3 = cw[:, 1:2] * (1.0 - oh1)
    den = (cw[:, 0:1] + cw[:, 1:2]) + 1e-8
    leaf_ref[...] = jnp.concatenate([c0, c1, c2, c3], axis=1) / den
    w0_ref[...] = cw[:, 0:1] / den
    w1_ref[...] = cw[:, 1:2] / den
    sl = pl.ds(i * BT, BT)
    ab0_ref[sl, :] = 1.0 - oh0
    ab1_ref[sl, :] = 1.0 - oh1

    tri = (lax.broadcasted_iota(jnp.int32, (BT, BT), 0)
           >= lax.broadcasted_iota(jnp.int32, (BT, BT), 1)).astype(
               jnp.float32)
    tc0 = lax.dot_general(tri, oh0, (((1,), (0,)), ((), ())),
                          preferred_element_type=jnp.float32)
    tc1 = lax.dot_general(tri, oh1, (((1,), (0,)), ((), ())),
                          preferred_element_type=jnp.float32)

    @pl.when(i == 0)
    def _init():
        for j in range(8):
            acc_ref[j] = 0.0

    cum00_ref[sl, :] = tc0 + acc_ref[6]
    cum10_ref[sl, :] = tc1 + acc_ref[7]
    acc_ref[6] += jnp.sum(oh0)
    acc_ref[7] += jnp.sum(oh1)

    acc_ref[0] += jnp.sum(ohc0)
    acc_ref[1] += jnp.sum(cw[:, 0:1])
    acc_ref[2] += jnp.sum(oh0)
    acc_ref[3] += jnp.sum(p00)
    acc_ref[4] += jnp.sum(oh1)
    acc_ref[5] += jnp.sum(p10)

    @pl.when(i == NT - 1)
    def _finalize():
        nb = jnp.float32(B)
        fc0 = acc_ref[0] / nb
        pc0 = acc_ref[1] / nb
        aux_c = 2.0 * (fc0 * pc0 + (1.0 - fc0) * (1.0 - pc0))
        f00 = acc_ref[2] / nb
        p00s = acc_ref[3] / nb
        f10 = acc_ref[4] / nb
        p10s = acc_ref[5] / nb
        aux_f = (2.0 * (f00 * p00s + (1.0 - f00) * (1.0 - p00s))
                 + 2.0 * (f10 * p10s + (1.0 - f10) * (1.0 - p10s)))
        aux_ref[...] = (AUX_COEF * (aux_c + aux_f / N_SUPER)).reshape(1, 1)

        tglob1 = (lax.broadcasted_iota(jnp.int32, (B, 1), 0)
                  .astype(jnp.float32) + 1.0)

        def route(ab, cum_ref, n0):
            a = ab[...]
            cum0 = cum_ref[...]
            n0p = jnp.floor((n0 + (BTE - 1)) * (1.0 / BTE)) * BTE
            cum1 = tglob1 - cum0
            pos = jnp.where(a == 0.0, cum0 - 1.0, n0p + cum1 - 1.0)
            return pos, n0p

        pos0, n0p0 = route(ab0_ref, cum00_ref, acc_ref[6])
        pos1, n0p1 = route(ab1_ref, cum10_ref, acc_ref[7])
        pos0_ref[...] = pos0.astype(jnp.int32)
        pos1_ref[...] = (pos1 + LP).astype(jnp.int32)

        t = lax.broadcasted_iota(jnp.int32, (1, 32), 1).astype(jnp.float32)
        e_g0 = jnp.where(t * BTE < n0p0, 0.0, 1.0)
        e_g1 = jnp.where((t - NTE // 2) * BTE < n0p1, 2.0, 3.0)
        eot = jnp.where(t < NTE // 2, e_g0,
                        jnp.where(t < NTE, e_g1, 0.0))
        eot_ref[...] = eot.astype(jnp.int32)


@functools.partial(
    pl.kernel, mesh=_mesh,
    out_type=jax.ShapeDtypeStruct((NSLOT, IN_DIM), jnp.float32),
    scratch_types=[
        pltpu.VMEM((TPW, IN_DIM), jnp.float32),
        pltpu.VMEM((TPW,), jnp.int32),
        pltpu.VMEM((TPW,), jnp.int32),
        pltpu.SemaphoreType.DMA,
    ],
)
def _sc_scatter(x_hbm, pos0_hbm, pos1_hbm, xp_hbm,
                rows_v, idx0_v, idx1_v, sem):
    wid = lax.axis_index("s") * 2 + lax.axis_index("c")
    base = wid * TPW
    pltpu.sync_copy(x_hbm.at[pl.ds(base, TPW)], rows_v)
    pltpu.sync_copy(pos0_hbm.at[pl.ds(base, TPW)], idx0_v)
    pltpu.sync_copy(pos1_hbm.at[pl.ds(base, TPW)], idx1_v)
    c0 = pltpu.async_copy(rows_v, xp_hbm.at[idx0_v], sem)
    c1 = pltpu.async_copy(rows_v, xp_hbm.at[idx1_v], sem)
    c0.wait()
    c1.wait()


def _expert_body(eot_ref, xp_ref, w1_ref, b1_ref, w2_ref, b2_ref,
                 g_ref, beta_ref, rhw_ref, chw_ref, prs_ref, drs_ref):
    x = xp_ref[...]
    hh = lax.dot_general(x, w1_ref[0], (((1,), (1,)), ((), ())),
                         preferred_element_type=jnp.float32)
    hh = hh + b1_ref[0]
    hh = 0.5 * hh * (1.0 + lax.erf(hh * (1.0 / math.sqrt(2.0))))
    eo = lax.dot_general(hh, w2_ref[0], (((1,), (1,)), ((), ())),
                         preferred_element_type=jnp.float32)
    eo = eo + b2_ref[0]
    mu = jnp.mean(eo, axis=1, keepdims=True)
    d = eo - mu
    var = jnp.mean(d * d, axis=1, keepdims=True)
    rstd = lax.rsqrt(var + 1e-5)
    eon = d * rstd * g_ref[0] + beta_ref[0]
    prs_ref[...] = lax.dot_general(eon, rhw_ref[...], (((1,), (1,)), ((), ())),
                                   preferred_element_type=jnp.float32)
    drs_ref[...] = lax.dot_general(eon, chw_ref[...], (((1,), (1,)), ((), ())),
                                   preferred_element_type=jnp.float32)


@functools.partial(
    pl.kernel, mesh=_mesh,
    out_type=[
        jax.ShapeDtypeStruct((B,), jnp.float32),
        jax.ShapeDtypeStruct((B,), jnp.float32),
    ],
    scratch_types=[
        pltpu.VMEM((TPW,), jnp.int32),
        pltpu.VMEM((TPW,), jnp.int32),
        pltpu.VMEM((TPW,), jnp.float32),
        pltpu.VMEM((TPW,), jnp.float32),
        pltpu.VMEM((32,), jnp.float32),
        pltpu.VMEM((TPW,), jnp.float32),
        pltpu.VMEM((TPW,), jnp.float32),
        pltpu.VMEM((TPW,), jnp.float32),
        pltpu.VMEM((TPW,), jnp.float32),
        pltpu.VMEM((TPW,), jnp.float32),
        pltpu.VMEM((TPW,), jnp.float32),
        pltpu.SemaphoreType.DMA,
    ],
)
def _sc_combine(prs_hbm, drs_hbm, pos0_hbm, pos1_hbm, w0_hbm, w1_hbm,
                bias_hbm, price_hbm, dir_hbm,
                idx0_v, idx1_v, w0_v, w1_v, bias_v,
                p0_v, p1_v, d0_v, d1_v, pout_v, dout_v, sem):
    wid = lax.axis_index("s") * 2 + lax.axis_index("c")
    base = wid * TPW
    pltpu.sync_copy(pos0_hbm.at[pl.ds(base, TPW)], idx0_v)
    pltpu.sync_copy(pos1_hbm.at[pl.ds(base, TPW)], idx1_v)
    pltpu.sync_copy(w0_hbm.at[pl.ds(base, TPW)], w0_v)
    pltpu.sync_copy(w1_hbm.at[pl.ds(base, TPW)], w1_v)
    pltpu.sync_copy(bias_hbm, bias_v)
    g0 = pltpu.async_copy(prs_hbm.at[idx0_v], p0_v, sem)
    g1 = pltpu.async_copy(prs_hbm.at[idx1_v], p1_v, sem)
    g2 = pltpu.async_copy(drs_hbm.at[idx0_v], d0_v, sem)
    g3 = pltpu.async_copy(drs_hbm.at[idx1_v], d1_v, sem)
    g0.wait()
    g1.wait()
    g2.wait()
    g3.wait()
    rb = bias_v[pl.ds(0, 16)]
    cb = bias_v[pl.ds(16, 16)]
    for j in range(TPW // 16):
        sl = pl.ds(j * 16, 16)
        a = w0_v[sl]
        bw = w1_v[sl]
        pout_v[sl] = a * p0_v[sl] + bw * p1_v[sl] + rb
        z = a * d0_v[sl] + bw * d1_v[sl] + cb
        dout_v[sl] = 1.0 / (1.0 + jnp.exp(-z))
    pltpu.sync_copy(pout_v, price_hbm.at[pl.ds(base, TPW)])
    pltpu.sync_copy(dout_v, dir_hbm.at[pl.ds(base, TPW)])


@jax.jit
def kernel(x, cg_w1, cg_b1, cg_w2, cg_b2, fg_w, fg_b, ex_w1, ex_b1,
           ex_w2, ex_b2, ex_g, ex_beta, rh_w, rh_b, ch_w, ch_b):
    f32 = jnp.float32
    fg_w2d = fg_w.reshape(E, IN_DIM + N_SUPER)
    leaf, aux, w0, w1, pos0, pos1, eot = pl.pallas_call(
        _gating_body,
        grid=(NT,),
        in_specs=[
            pl.BlockSpec((BT, IN_DIM), lambda i: (i, 0)),
            pl.BlockSpec((IN_DIM // 2, IN_DIM), lambda i: (0, 0)),
            pl.BlockSpec((1, IN_DIM // 2), lambda i: (0, 0)),
            pl.BlockSpec((N_SUPER, IN_DIM // 2), lambda i: (0, 0)),
            pl.BlockSpec((1, N_SUPER), lambda i: (0, 0)),
            pl.BlockSpec((E, IN_DIM + N_SUPER), lambda i: (0, 0)),
            pl.BlockSpec((1, E), lambda i: (0, 0)),
        ],
        out_specs=[
            pl.BlockSpec((BT, E), lambda i: (i, 0)),
            pl.BlockSpec((1, 1), lambda i: (0, 0)),
            pl.BlockSpec((BT, 1), lambda i: (i, 0)),
            pl.BlockSpec((BT, 1), lambda i: (i, 0)),
            pl.BlockSpec((B, 1), lambda i: (0, 0)),
            pl.BlockSpec((B, 1), lambda i: (0, 0)),
            pl.BlockSpec((1, 32), lambda i: (0, 0)),
        ],
        out_shape=[
            jax.ShapeDtypeStruct((B, E), f32),
            jax.ShapeDtypeStruct((1, 1), f32),
            jax.ShapeDtypeStruct((B, 1), f32),
            jax.ShapeDtypeStruct((B, 1), f32),
            jax.ShapeDtypeStruct((B, 1), jnp.int32),
            jax.ShapeDtypeStruct((B, 1), jnp.int32),
            jax.ShapeDtypeStruct((1, 32), jnp.int32),
        ],
        scratch_shapes=[
            pltpu.SMEM((8,), f32),
            pltpu.VMEM((B, 1), f32),
            pltpu.VMEM((B, 1), f32),
            pltpu.VMEM((B, 1), f32),
            pltpu.VMEM((B, 1), f32),
        ],
    )(x, cg_w1, cg_b1.reshape(1, -1), cg_w2, cg_b2.reshape(1, -1),
      fg_w2d, fg_b.reshape(1, E))

    pos0_1 = pos0.reshape(B)
    pos1_1 = pos1.reshape(B)
    x_perm = _sc_scatter(x, pos0_1, pos1_1)

    grid_spec = pltpu.PrefetchScalarGridSpec(
        num_scalar_prefetch=1,
        grid=(NTE,),
        in_specs=[
            pl.BlockSpec((BTE, IN_DIM), lambda t, eot: (t, 0)),
            pl.BlockSpec((1, HID, IN_DIM), lambda t, eot: (eot[t], 0, 0)),
            pl.BlockSpec((1, 1, HID), lambda t, eot: (eot[t], 0, 0)),
            pl.BlockSpec((1, OUT, HID), lambda t, eot: (eot[t], 0, 0)),
            pl.BlockSpec((1, 1, OUT), lambda t, eot: (eot[t], 0, 0)),
            pl.BlockSpec((1, 1, OUT), lambda t, eot: (eot[t], 0, 0)),
            pl.BlockSpec((1, 1, OUT), lambda t, eot: (eot[t], 0, 0)),
            pl.BlockSpec((1, OUT), lambda t, eot: (0, 0)),
            pl.BlockSpec((1, OUT), lambda t, eot: (0, 0)),
        ],
        out_specs=[
            pl.BlockSpec((BTE, 1), lambda t, eot: (t, 0)),
            pl.BlockSpec((BTE, 1), lambda t, eot: (t, 0)),
        ],
    )
    prs, drs = pl.pallas_call(
        _expert_body,
        grid_spec=grid_spec,
        out_shape=[
            jax.ShapeDtypeStruct((NSLOT, 1), f32),
            jax.ShapeDtypeStruct((NSLOT, 1), f32),
        ],
    )(eot.reshape(32), x_perm, ex_w1, ex_b1.reshape(E, 1, HID), ex_w2,
      ex_b2.reshape(E, 1, OUT), ex_g.reshape(E, 1, OUT),
      ex_beta.reshape(E, 1, OUT), rh_w, ch_w)

    bias_arr = jnp.concatenate([
        jnp.broadcast_to(rh_b.reshape(1), (16,)),
        jnp.broadcast_to(ch_b.reshape(1), (16,)),
    ]).astype(f32)
    price, direction = _sc_combine(
        prs.reshape(NSLOT), drs.reshape(NSLOT), pos0_1, pos1_1,
        w0.reshape(B), w1.reshape(B), bias_arr)

    return price.reshape(B, 1), direction.reshape(B, 1), leaf, aux.reshape(())


# speedup vs baseline: 1.3573x; 1.0477x over previous
import functools
import math

import jax
import jax.numpy as jnp
from jax import lax
from jax.experimental import pallas as pl
from jax.experimental.pallas import tpu as pltpu
from jax.experimental.pallas import tpu_sc as plsc

B = 2048
IN_DIM = 1024
N_SUPER = 2
N_SUB = 2
E = 4
HID = 2048
OUT = 512
AUX_COEF = 0.01

BT = 512
NT = B // BT
BTE = 512
LP = B + BTE
NSLOT = 2 * LP
NTE = NSLOT // BTE

NW = 32
TPW = B // NW

_mesh = plsc.VectorSubcoreMesh(core_axis_name="c", subcore_axis_name="s")


def _gating_body(x_ref, cgw1_ref, cgb1_ref, cgw2_ref, cgb2_ref,
                 fgw_ref, fgb_ref,
                 leaf_ref, aux_ref, w0_ref, w1_ref, pos0_ref, pos1_ref,
                 eot_ref, xpack_ref, acc_ref, ab0_ref, ab1_ref, cum00_ref,
                 cum10_ref):
    i = pl.program_id(0)
    x = x_ref[...]
    xb16 = x.astype(jnp.bfloat16)
    lo_u = lax.bitcast_convert_type(xb16[:, :IN_DIM // 2],
                                    jnp.uint16).astype(jnp.uint32)
    hi_u = lax.bitcast_convert_type(xb16[:, IN_DIM // 2:],
                                    jnp.uint16).astype(jnp.uint32)
    xpack_ref[...] = lax.bitcast_convert_type(lo_u | (hi_u << 16), jnp.int32)
    h = lax.dot_general(x, cgw1_ref[...], (((1,), (1,)), ((), ())),
                        preferred_element_type=jnp.float32)
    h = jnp.maximum(h + cgb1_ref[...], 0.0)
    cl = lax.dot_general(h, cgw2_ref[...], (((1,), (1,)), ((), ())),
                         preferred_element_type=jnp.float32)
    cl = cl + cgb2_ref[...]
    m = jnp.max(cl, axis=1, keepdims=True)
    ex = jnp.exp(cl - m)
    cw = ex / jnp.sum(ex, axis=1, keepdims=True)
    ohc0 = (cl[:, 0:1] >= cl[:, 1:2]).astype(jnp.float32)

    x_aug = jnp.concatenate([x, cw], axis=1)
    fl = (lax.dot_general(x_aug, fgw_ref[...], (((1,), (1,)), ((), ())),
                          preferred_element_type=jnp.float32)
          + fgb_ref[...])
    oh0 = (fl[:, 0:1] >= fl[:, 1:2]).astype(jnp.float32)
    oh1 = (fl[:, 2:3] >= fl[:, 3:4]).astype(jnp.float32)

    m0 = jnp.maximum(fl[:, 0:1], fl[:, 1:2])
    p00 = jnp.exp(fl[:, 0:1] - m0) / (jnp.exp(fl[:, 0:1] - m0)
                                      + jnp.exp(fl[:, 1:2] - m0))
    m1 = jnp.maximum(fl[:, 2:3], fl[:, 3:4])
    p10 = jnp.exp(fl[:, 2:3] - m1) / (jnp.exp(fl[:, 2:3] - m1)
                                      + jnp.exp(fl[:, 3:4] - m1))

    c0 = cw[:, 0:1] * oh0
    c1 = cw[:, 0:1] * (1.0 - oh0)
    c2 = cw[:, 1:2] * oh1
    c3 = cw[:, 1:2] * (1.0 - oh1)
    den = (cw[:, 0:1] + cw[:, 1:2]) + 1e-8
    leaf_ref[...] = jnp.concatenate([c0, c1, c2, c3], axis=1) / den
    w0_ref[...] = cw[:, 0:1] / den
    w1_ref[...] = cw[:, 1:2] / den
    sl = pl.ds(i * BT, BT)
    ab0_ref[sl, :] = 1.0 - oh0
    ab1_ref[sl, :] = 1.0 - oh1

    tri = (lax.broadcasted_iota(jnp.int32, (BT, BT), 0)
           >= lax.broadcasted_iota(jnp.int32, (BT, BT), 1)).astype(
               jnp.float32)
    tc0 = lax.dot_general(tri, oh0, (((1,), (0,)), ((), ())),
                          preferred_element_type=jnp.float32)
    tc1 = lax.dot_general(tri, oh1, (((1,), (0,)), ((), ())),
                          preferred_element_type=jnp.float32)

    @pl.when(i == 0)
    def _init():
        for j in range(8):
            acc_ref[j] = 0.0

    cum00_ref[sl, :] = tc0 + acc_ref[6]
    cum10_ref[sl, :] = tc1 + acc_ref[7]
    acc_ref[6] += jnp.sum(oh0)
    acc_ref[7] += jnp.sum(oh1)

    acc_ref[0] += jnp.sum(ohc0)
    acc_ref[1] += jnp.sum(cw[:, 0:1])
    acc_ref[2] += jnp.sum(oh0)
    acc_ref[3] += jnp.sum(p00)
    acc_ref[4] += jnp.sum(oh1)
    acc_ref[5] += jnp.sum(p10)

    @pl.when(i == NT - 1)
    def _finalize():
        nb = jnp.float32(B)
        fc0 = acc_ref[0] / nb
        pc0 = acc_ref[1] / nb
        aux_c = 2.0 * (fc0 * pc0 + (1.0 - fc0) * (1.0 - pc0))
        f00 = acc_ref[2] / nb
        p00s = acc_ref[3] / nb
        f10 = acc_ref[4] / nb
        p10s = acc_ref[5] / nb
        aux_f = (2.0 * (f00 * p00s + (1.0 - f00) * (1.0 - p00s))
                 + 2.0 * (f10 * p10s + (1.0 - f10) * (1.0 - p10s)))
        aux_ref[...] = (AUX_COEF * (aux_c + aux_f / N_SUPER)).reshape(1, 1)

        tglob1 = (lax.broadcasted_iota(jnp.int32, (B, 1), 0)
                  .astype(jnp.float32) + 1.0)

        def route(ab, cum_ref, n0):
            a = ab[...]
            cum0 = cum_ref[...]
            n0p = jnp.floor((n0 + (BTE - 1)) * (1.0 / BTE)) * BTE
            cum1 = tglob1 - cum0
            pos = jnp.where(a == 0.0, cum0 - 1.0, n0p + cum1 - 1.0)
            return pos, n0p

        pos0, n0p0 = route(ab0_ref, cum00_ref, acc_ref[6])
        pos1, n0p1 = route(ab1_ref, cum10_ref, acc_ref[7])
        pos0_ref[...] = pos0.astype(jnp.int32)
        pos1_ref[...] = (pos1 + LP).astype(jnp.int32)

        t = lax.broadcasted_iota(jnp.int32, (1, 32), 1).astype(jnp.float32)
        e_g0 = jnp.where(t * BTE < n0p0, 0.0, 1.0)
        e_g1 = jnp.where((t - NTE // 2) * BTE < n0p1, 2.0, 3.0)
        eot = jnp.where(t < NTE // 2, e_g0,
                        jnp.where(t < NTE, e_g1, 0.0))
        eot_ref[...] = eot.astype(jnp.int32)


@functools.partial(
    pl.kernel, mesh=_mesh,
    out_type=jax.ShapeDtypeStruct((NSLOT, IN_DIM // 2), jnp.int32),
    scratch_types=[
        pltpu.VMEM((TPW, IN_DIM // 2), jnp.int32),
        pltpu.VMEM((TPW,), jnp.int32),
        pltpu.VMEM((TPW,), jnp.int32),
        pltpu.SemaphoreType.DMA,
    ],
)
def _sc_scatter(x_hbm, pos0_hbm, pos1_hbm, xp_hbm,
                rows_v, idx0_v, idx1_v, sem):
    wid = lax.axis_index("s") * 2 + lax.axis_index("c")
    base = wid * TPW
    pltpu.sync_copy(x_hbm.at[pl.ds(base, TPW)], rows_v)
    pltpu.sync_copy(pos0_hbm.at[pl.ds(base, TPW)], idx0_v)
    pltpu.sync_copy(pos1_hbm.at[pl.ds(base, TPW)], idx1_v)
    c0 = pltpu.async_copy(rows_v, xp_hbm.at[idx0_v], sem)
    c1 = pltpu.async_copy(rows_v, xp_hbm.at[idx1_v], sem)
    c0.wait()
    c1.wait()


def _expert_body(eot_ref, xp_ref, w1_ref, b1_ref, w2_ref, b2_ref,
                 g_ref, beta_ref, rhw_ref, chw_ref, prs_ref, drs_ref):
    u = lax.bitcast_convert_type(xp_ref[...], jnp.uint32)
    lo = lax.bitcast_convert_type((u & 0xffff).astype(jnp.uint16),
                                  jnp.bfloat16)
    hi = lax.bitcast_convert_type((u >> 16).astype(jnp.uint16),
                                  jnp.bfloat16)
    x = jnp.concatenate([lo, hi], axis=1).astype(jnp.float32)
    hh = lax.dot_general(x, w1_ref[0], (((1,), (1,)), ((), ())),
                         preferred_element_type=jnp.float32)
    hh = hh + b1_ref[0]
    hh = 0.5 * hh * (1.0 + lax.erf(hh * (1.0 / math.sqrt(2.0))))
    eo = lax.dot_general(hh, w2_ref[0], (((1,), (1,)), ((), ())),
                         preferred_element_type=jnp.float32)
    eo = eo + b2_ref[0]
    mu = jnp.mean(eo, axis=1, keepdims=True)
    d = eo - mu
    var = jnp.mean(d * d, axis=1, keepdims=True)
    rstd = lax.rsqrt(var + 1e-5)
    eon = d * rstd * g_ref[0] + beta_ref[0]
    prs_ref[...] = lax.dot_general(eon, rhw_ref[...], (((1,), (1,)), ((), ())),
                                   preferred_element_type=jnp.float32)
    drs_ref[...] = lax.dot_general(eon, chw_ref[...], (((1,), (1,)), ((), ())),
                                   preferred_element_type=jnp.float32)


@functools.partial(
    pl.kernel, mesh=_mesh,
    out_type=[
        jax.ShapeDtypeStruct((B,), jnp.float32),
        jax.ShapeDtypeStruct((B,), jnp.float32),
    ],
    scratch_types=[
        pltpu.VMEM((TPW,), jnp.int32),
        pltpu.VMEM((TPW,), jnp.int32),
        pltpu.VMEM((TPW,), jnp.float32),
        pltpu.VMEM((TPW,), jnp.float32),
        pltpu.VMEM((32,), jnp.float32),
        pltpu.VMEM((TPW,), jnp.float32),
        pltpu.VMEM((TPW,), jnp.float32),
        pltpu.VMEM((TPW,), jnp.float32),
        pltpu.VMEM((TPW,), jnp.float32),
        pltpu.VMEM((TPW,), jnp.float32),
        pltpu.VMEM((TPW,), jnp.float32),
        pltpu.SemaphoreType.DMA,
    ],
)
def _sc_combine(prs_hbm, drs_hbm, pos0_hbm, pos1_hbm, w0_hbm, w1_hbm,
                bias_hbm, price_hbm, dir_hbm,
                idx0_v, idx1_v, w0_v, w1_v, bias_v,
                p0_v, p1_v, d0_v, d1_v, pout_v, dout_v, sem):
    wid = lax.axis_index("s") * 2 + lax.axis_index("c")
    base = wid * TPW
    pltpu.sync_copy(pos0_hbm.at[pl.ds(base, TPW)], idx0_v)
    pltpu.sync_copy(pos1_hbm.at[pl.ds(base, TPW)], idx1_v)
    pltpu.sync_copy(w0_hbm.at[pl.ds(base, TPW)], w0_v)
    pltpu.sync_copy(w1_hbm.at[pl.ds(base, TPW)], w1_v)
    pltpu.sync_copy(bias_hbm, bias_v)
    g0 = pltpu.async_copy(prs_hbm.at[idx0_v], p0_v, sem)
    g1 = pltpu.async_copy(prs_hbm.at[idx1_v], p1_v, sem)
    g2 = pltpu.async_copy(drs_hbm.at[idx0_v], d0_v, sem)
    g3 = pltpu.async_copy(drs_hbm.at[idx1_v], d1_v, sem)
    g0.wait()
    g1.wait()
    g2.wait()
    g3.wait()
    rb = bias_v[pl.ds(0, 16)]
    cb = bias_v[pl.ds(16, 16)]
    for j in range(TPW // 16):
        sl = pl.ds(j * 16, 16)
        a = w0_v[sl]
        bw = w1_v[sl]
        pout_v[sl] = a * p0_v[sl] + bw * p1_v[sl] + rb
        z = a * d0_v[sl] + bw * d1_v[sl] + cb
        dout_v[sl] = 1.0 / (1.0 + jnp.exp(-z))
    pltpu.sync_copy(pout_v, price_hbm.at[pl.ds(base, TPW)])
    pltpu.sync_copy(dout_v, dir_hbm.at[pl.ds(base, TPW)])


@jax.jit
def kernel(x, cg_w1, cg_b1, cg_w2, cg_b2, fg_w, fg_b, ex_w1, ex_b1,
           ex_w2, ex_b2, ex_g, ex_beta, rh_w, rh_b, ch_w, ch_b):
    f32 = jnp.float32
    fg_w2d = fg_w.reshape(E, IN_DIM + N_SUPER)
    leaf, aux, w0, w1, pos0, pos1, eot, xpack = pl.pallas_call(
        _gating_body,
        grid=(NT,),
        in_specs=[
            pl.BlockSpec((BT, IN_DIM), lambda i: (i, 0)),
            pl.BlockSpec((IN_DIM // 2, IN_DIM), lambda i: (0, 0)),
            pl.BlockSpec((1, IN_DIM // 2), lambda i: (0, 0)),
            pl.BlockSpec((N_SUPER, IN_DIM // 2), lambda i: (0, 0)),
            pl.BlockSpec((1, N_SUPER), lambda i: (0, 0)),
            pl.BlockSpec((E, IN_DIM + N_SUPER), lambda i: (0, 0)),
            pl.BlockSpec((1, E), lambda i: (0, 0)),
        ],
        out_specs=[
            pl.BlockSpec((BT, E), lambda i: (i, 0)),
            pl.BlockSpec((1, 1), lambda i: (0, 0)),
            pl.BlockSpec((BT, 1), lambda i: (i, 0)),
            pl.BlockSpec((BT, 1), lambda i: (i, 0)),
            pl.BlockSpec((B, 1), lambda i: (0, 0)),
            pl.BlockSpec((B, 1), lambda i: (0, 0)),
            pl.BlockSpec((1, 32), lambda i: (0, 0)),
            pl.BlockSpec((BT, IN_DIM // 2), lambda i: (i, 0)),
        ],
        out_shape=[
            jax.ShapeDtypeStruct((B, E), f32),
            jax.ShapeDtypeStruct((1, 1), f32),
            jax.ShapeDtypeStruct((B, 1), f32),
            jax.ShapeDtypeStruct((B, 1), f32),
            jax.ShapeDtypeStruct((B, 1), jnp.int32),
            jax.ShapeDtypeStruct((B, 1), jnp.int32),
            jax.ShapeDtypeStruct((1, 32), jnp.int32),
            jax.ShapeDtypeStruct((B, IN_DIM // 2), jnp.int32),
        ],
        scratch_shapes=[
            pltpu.SMEM((8,), f32),
            pltpu.VMEM((B, 1), f32),
            pltpu.VMEM((B, 1), f32),
            pltpu.VMEM((B, 1), f32),
            pltpu.VMEM((B, 1), f32),
        ],
    )(x, cg_w1, cg_b1.reshape(1, -1), cg_w2, cg_b2.reshape(1, -1),
      fg_w2d, fg_b.reshape(1, E))

    pos0_1 = pos0.reshape(B)
    pos1_1 = pos1.reshape(B)
    x_perm = _sc_scatter(xpack, pos0_1, pos1_1)

    grid_spec = pltpu.PrefetchScalarGridSpec(
        num_scalar_prefetch=1,
        grid=(NTE,),
        in_specs=[
            pl.BlockSpec((BTE, IN_DIM // 2), lambda t, eot: (t, 0)),
            pl.BlockSpec((1, HID, IN_DIM), lambda t, eot: (eot[t], 0, 0)),
            pl.BlockSpec((1, 1, HID), lambda t, eot: (eot[t], 0, 0)),
            pl.BlockSpec((1, OUT, HID), lambda t, eot: (eot[t], 0, 0)),
            pl.BlockSpec((1, 1, OUT), lambda t, eot: (eot[t], 0, 0)),
            pl.BlockSpec((1, 1, OUT), lambda t, eot: (eot[t], 0, 0)),
            pl.BlockSpec((1, 1, OUT), lambda t, eot: (eot[t], 0, 0)),
            pl.BlockSpec((1, OUT), lambda t, eot: (0, 0)),
            pl.BlockSpec((1, OUT), lambda t, eot: (0, 0)),
        ],
        out_specs=[
            pl.BlockSpec((BTE, 1), lambda t, eot: (t, 0)),
            pl.BlockSpec((BTE, 1), lambda t, eot: (t, 0)),
        ],
    )
    prs, drs = pl.pallas_call(
        _expert_body,
        grid_spec=grid_spec,
        out_shape=[
            jax.ShapeDtypeStruct((NSLOT, 1), f32),
            jax.ShapeDtypeStruct((NSLOT, 1), f32),
        ],
    )(eot.reshape(32), x_perm, ex_w1, ex_b1.reshape(E, 1, HID), ex_w2,
      ex_b2.reshape(E, 1, OUT), ex_g.reshape(E, 1, OUT),
      ex_beta.reshape(E, 1, OUT), rh_w, ch_w)

    bias_arr = jnp.concatenate([
        jnp.broadcast_to(rh_b.reshape(1), (16,)),
        jnp.broadcast_to(ch_b.reshape(1), (16,)),
    ]).astype(f32)
    price, direction = _sc_combine(
        prs.reshape(NSLOT), drs.reshape(NSLOT), pos0_1, pos1_1,
        w0.reshape(B), w1.reshape(B), bias_arr)

    return price.reshape(B, 1), direction.reshape(B, 1), leaf, aux.reshape(())
